# Initial kernel scaffold; baseline (speedup 1.0000x reference)
import jax
import jax.numpy as jnp
from jax.experimental import pallas as pl


def kernel(coords, tokens, q0, q1, q2):
    raise NotImplementedError("write your pallas kernel here")


# speedup vs baseline: 5.7180x; 5.7180x over previous
import functools

import jax
import jax.numpy as jnp
from jax import lax
from jax.experimental import pallas as pl
from jax.experimental.pallas import tpu as pltpu
from jax.experimental.pallas import tpu_sc as plsc

_B, _N, _C = 8, 16384, 128
_NB16, _NB8, _NB4 = 4096, 512, 64
_T0, _T1, _T2 = 16, 32, 64
_NSC, _NTEC = 2, 16
_BPC = _B // _NSC
_PPT = _N // _NTEC
_CH = 128
_NCH = _PPT // _CH
_RPT16 = _NB16 // _NTEC
_RPT8 = _NB8 // _NTEC


def _voxel_idx_body(coords_ref, out_ref):
    cxyz = coords_ref[0]
    mins = jnp.min(cxyz, axis=1, keepdims=True)
    maxs = jnp.max(cxyz, axis=1, keepdims=True)
    denom = jnp.maximum(maxs - mins, 1e-6)
    normalized = (cxyz - mins) / denom
    cell = jnp.clip(jnp.floor(normalized * 16.0), 0.0, 15.0).astype(jnp.int32)
    row = lax.broadcasted_iota(jnp.int32, (3, _N), 0)
    w = jnp.where(row == 0, 256, jnp.where(row == 1, 16, 1))
    out_ref[0] = jnp.sum(cell * w, axis=0, keepdims=True)


def _voxel_idx(coords_t):
    return pl.pallas_call(
        _voxel_idx_body,
        grid=(_B,),
        in_specs=[pl.BlockSpec((1, 3, _N), lambda b: (b, 0, 0))],
        out_specs=pl.BlockSpec((1, 1, _N), lambda b: (b, 0, 0)),
        out_shape=jax.ShapeDtypeStruct((_B, 1, _N), jnp.int32),
    )(coords_t)


def _sc_body(tok_hbm, idx_hbm, p8_hbm, p4_hbm,
             s16_hbm, c16_hbm, s8_hbm, c8_hbm, s4_hbm, c4_hbm,
             tok_v, idx_v, ones_v, zero_v, pidx8_v, pidx4_v,
             acc16, acc8, acc4):
    cid = lax.axis_index("c")
    sid = lax.axis_index("s")

    zvec = jnp.zeros((16,), jnp.float32)
    ovec = jnp.ones((16,), jnp.float32)

    @pl.loop(0, 16)
    def _fill_zero(r):
        for g in range(_C // 16):
            zero_v[r, pl.ds(g * 16, 16)] = zvec

    @pl.loop(0, _CH)
    def _fill_ones(r):
        for g in range(_C // 16):
            ones_v[r, pl.ds(g * 16, 16)] = ovec

    pltpu.sync_copy(p8_hbm.at[pl.ds(sid * 8, 2)], pidx8_v)
    pltpu.sync_copy(p4_hbm.at[pl.ds(sid * 8, 2)], pidx4_v)

    def zero_tables():
        for j in range(_RPT16 // 16):
            pltpu.sync_copy(zero_v, acc16.at[pl.ds(sid * _RPT16 + j * 16, 16)])
        for j in range(_RPT8 // 16):
            pltpu.sync_copy(zero_v, acc8.at[pl.ds(sid * _RPT8 + j * 16, 16)])

        @pl.when(sid == 0)
        def _zero_l4():
            for j in range(_NB4 // 16):
                pltpu.sync_copy(zero_v, acc4.at[pl.ds(j * 16, 16)])

    def pool_and_out(b, fine_hbm, coarse8_hbm, coarse4_hbm):
        pltpu.sync_copy(acc16.at[pl.ds(sid * _RPT16, _RPT16)],
                        fine_hbm.at[pl.ds(b * _NB16 + sid * _RPT16, _RPT16)])
        for j in range(2):
            pltpu.sync_copy(
                fine_hbm.at[pl.ds(b * _NB16 + sid * _RPT16 + j * _CH, _CH)],
                tok_v)
            pltpu.sync_copy(tok_v, acc8.at[pidx8_v.at[j]], add=True)
            pltpu.sync_copy(tok_v, acc4.at[pidx4_v.at[j]], add=True)
        plsc.subcore_barrier()
        pltpu.sync_copy(acc8.at[pl.ds(sid * _RPT8, _RPT8)],
                        coarse8_hbm.at[pl.ds(b * _NB8 + sid * _RPT8, _RPT8)])

        @pl.when(sid == 0)
        def _out_l4():
            pltpu.sync_copy(acc4, coarse4_hbm.at[pl.ds(b * _NB4, _NB4)])

    for k in range(_BPC):
        b = cid * _BPC + k

        zero_tables()
        plsc.subcore_barrier()
        pltpu.sync_copy(idx_hbm.at[pl.ds(b * (_N // _CH) + sid * _NCH, _NCH)],
                        idx_v)
        pt0 = b * _N + sid * _PPT
        for j in range(_NCH):
            pltpu.sync_copy(tok_hbm.at[pl.ds(pt0 + j * _CH, _CH)], tok_v)
            pltpu.sync_copy(tok_v, acc16.at[idx_v.at[j]], add=True)
        plsc.subcore_barrier()
        pool_and_out(b, s16_hbm, s8_hbm, s4_hbm)

        zero_tables()
        plsc.subcore_barrier()
        for j in range(_NCH):
            pltpu.sync_copy(ones_v, acc16.at[idx_v.at[j]], add=True)
        plsc.subcore_barrier()
        pool_and_out(b, c16_hbm, c8_hbm, c4_hbm)


def _parent_tables():
    r = jnp.arange(_NB16, dtype=jnp.int32)
    p8 = ((r >> 9) & 7) * 64 + ((r >> 5) & 7) * 8 + ((r >> 1) & 7)
    p4 = ((r >> 10) & 3) * 16 + ((r >> 6) & 3) * 4 + ((r >> 2) & 3)
    def pad(tab):
        tab = tab.reshape(_NTEC, 2, _CH)
        return jnp.pad(tab, ((0, 0), (0, 6), (0, 0))).reshape(_NTEC * 8, _CH)
    return pad(p8), pad(p4)


def _sc_segment_sum(tok2d, idx2d):
    p8_tab, p4_tab = _parent_tables()
    mesh = plsc.VectorSubcoreMesh(core_axis_name="c", subcore_axis_name="s")
    out_type = (
        jax.ShapeDtypeStruct((_B * _NB16, _C), jnp.float32),
        jax.ShapeDtypeStruct((_B * _NB16, _C), jnp.float32),
        jax.ShapeDtypeStruct((_B * _NB8, _C), jnp.float32),
        jax.ShapeDtypeStruct((_B * _NB8, _C), jnp.float32),
        jax.ShapeDtypeStruct((_B * _NB4, _C), jnp.float32),
        jax.ShapeDtypeStruct((_B * _NB4, _C), jnp.float32),
    )
    scratch = [
        pltpu.VMEM((_CH, _C), jnp.float32),
        pltpu.VMEM((_NCH, _CH), jnp.int32),
        pltpu.VMEM((_CH, _C), jnp.float32),
        pltpu.VMEM((16, _C), jnp.float32),
        pltpu.VMEM((2, _CH), jnp.int32),
        pltpu.VMEM((2, _CH), jnp.int32),
        pltpu.VMEM_SHARED((_NB16, _C), jnp.float32),
        pltpu.VMEM_SHARED((_NB8, _C), jnp.float32),
        pltpu.VMEM_SHARED((_NB4, _C), jnp.float32),
    ]
    return pl.kernel(
        _sc_body,
        out_type=out_type,
        mesh=mesh,
        scratch_types=scratch,
    )(tok2d, idx2d, p8_tab, p4_tab)


def _attn_body(s16_ref, c16_ref, s8_ref, c8_ref, s4_ref, c4_ref,
               q0_ref, q1_ref, q2_ref, out_ref):
    scale = jnp.float32(_C ** -0.5)

    def level(s_ref, c_ref, q_ref, t0, t_len):
        sums = s_ref[0]
        cnt_tab = c_ref[0]
        q = q_ref[...]
        count = cnt_tab[:, 0:1]
        feats = sums / jnp.maximum(count, 1.0)
        logits = lax.dot_general(
            q, feats, (((1,), (1,)), ((), ())),
            preferred_element_type=jnp.float32,
            precision=lax.Precision.HIGHEST) * scale
        ones_t = jnp.full((t_len, _C), 1.0 / _C, jnp.float32)
        cnt_row = lax.dot_general(
            ones_t, cnt_tab, (((1,), (1,)), ((), ())),
            preferred_element_type=jnp.float32,
            precision=lax.Precision.HIGHEST)
        logits = jnp.where(cnt_row > 0.0, logits, jnp.float32(-1e30))
        m = jnp.max(logits, axis=1, keepdims=True)
        e = jnp.exp(logits - m)
        attn = e / jnp.sum(e, axis=1, keepdims=True)
        ctx = lax.dot_general(
            attn, feats, (((1,), (0,)), ((), ())),
            preferred_element_type=jnp.float32,
            precision=lax.Precision.HIGHEST)
        out_ref[0, t0:t0 + t_len, :] = ctx

    level(s4_ref, c4_ref, q0_ref, 0, _T0)
    level(s8_ref, c8_ref, q1_ref, _T0, _T1)
    level(s16_ref, c16_ref, q2_ref, _T0 + _T1, _T2)


def _attn_pool(s16, c16, s8, c8, s4, c4, q0, q1, q2):
    return pl.pallas_call(
        _attn_body,
        grid=(_B,),
        in_specs=[
            pl.BlockSpec((1, _NB16, _C), lambda b: (b, 0, 0)),
            pl.BlockSpec((1, _NB16, _C), lambda b: (b, 0, 0)),
            pl.BlockSpec((1, _NB8, _C), lambda b: (b, 0, 0)),
            pl.BlockSpec((1, _NB8, _C), lambda b: (b, 0, 0)),
            pl.BlockSpec((1, _NB4, _C), lambda b: (b, 0, 0)),
            pl.BlockSpec((1, _NB4, _C), lambda b: (b, 0, 0)),
            pl.BlockSpec((_T0, _C), lambda b: (0, 0)),
            pl.BlockSpec((_T1, _C), lambda b: (0, 0)),
            pl.BlockSpec((_T2, _C), lambda b: (0, 0)),
        ],
        out_specs=pl.BlockSpec((1, _T0 + _T1 + _T2, _C), lambda b: (b, 0, 0)),
        out_shape=jax.ShapeDtypeStruct((_B, _T0 + _T1 + _T2, _C), jnp.float32),
    )(s16, c16, s8, c8, s4, c4, q0, q1, q2)


def kernel(coords, tokens, q0, q1, q2):
    coords_t = jnp.transpose(coords, (0, 2, 1))
    idx = _voxel_idx(coords_t)
    idx2d = idx.reshape(_B * _N // _CH, _CH)
    tok2d = tokens.reshape(_B * _N, _C)
    s16, c16, s8, c8, s4, c4 = _sc_segment_sum(tok2d, idx2d)
    return _attn_pool(
        s16.reshape(_B, _NB16, _C), c16.reshape(_B, _NB16, _C),
        s8.reshape(_B, _NB8, _C), c8.reshape(_B, _NB8, _C),
        s4.reshape(_B, _NB4, _C), c4.reshape(_B, _NB4, _C),
        q0, q1, q2)
